# baseline (speedup 1.0000x reference)
import functools

import jax
import jax.numpy as jnp
from jax import lax
from jax.experimental import pallas as pl
from jax.experimental.pallas import tpu as pltpu
from jax.experimental.pallas import tpu_sc as plsc

SEQ_LEN = 200
D_MODEL = 128
BATCH = 4096
NC = 2
NS = 16
NW = NC * NS
RPW = BATCH // NW
DEPTH = 4


def _sc_body(table_hbm, idx_hbm, out_hbm, idx_v, rows_v, gsem, wsem):
    wid = lax.axis_index("s") * NC + lax.axis_index("c")
    base = wid * RPW

    pltpu.sync_copy(idx_hbm, idx_v)
    pltpu.make_async_copy(
        table_hbm.at[idx_v.at[pl.ds(0, 128)]], rows_v.at[pl.ds(0, 128)], gsem
    ).start()
    pltpu.make_async_copy(
        table_hbm.at[idx_v.at[pl.ds(128, 72)]], rows_v.at[pl.ds(128, 72)], gsem
    ).start()
    pltpu.make_async_copy(
        table_hbm.at[idx_v.at[pl.ds(0, 128)]], rows_v.at[pl.ds(0, 128)], gsem
    ).wait()
    pltpu.make_async_copy(
        table_hbm.at[idx_v.at[pl.ds(128, 72)]], rows_v.at[pl.ds(128, 72)], gsem
    ).wait()

    def _out_copy(j):
        return pltpu.make_async_copy(
            rows_v, out_hbm.at[pl.ds((base + j) * SEQ_LEN, SEQ_LEN)], wsem
        )

    for j in range(DEPTH):
        _out_copy(j).start()

    def _ring(j, carry):
        _out_copy(j + DEPTH).start()
        _out_copy(j).wait()
        return carry

    lax.fori_loop(0, RPW - DEPTH, _ring, 0)

    for j in range(RPW - DEPTH, RPW):
        _out_copy(j).wait()


@jax.jit
def _run(pos_embed, positions):
    idx = positions.astype(jnp.int32)
    sc_kernel = pl.kernel(
        _sc_body,
        out_type=jax.ShapeDtypeStruct((BATCH * SEQ_LEN, D_MODEL), jnp.float32),
        mesh=plsc.VectorSubcoreMesh(core_axis_name="c", subcore_axis_name="s"),
        scratch_types=[
            pltpu.VMEM((SEQ_LEN,), jnp.int32),
            pltpu.VMEM((SEQ_LEN, D_MODEL), jnp.float32),
            pltpu.SemaphoreType.DMA,
            pltpu.SemaphoreType.DMA,
        ],
    )
    out2d = sc_kernel(pos_embed, idx)
    return out2d.reshape(BATCH, SEQ_LEN, D_MODEL)


def kernel(batch_size, pos_embed, positions):
    return _run(pos_embed, positions)


# speedup vs baseline: 1.0685x; 1.0685x over previous
import functools

import jax
import jax.numpy as jnp
from jax import lax
from jax.experimental import pallas as pl
from jax.experimental.pallas import tpu as pltpu
from jax.experimental.pallas import tpu_sc as plsc

SEQ_LEN = 200
D_MODEL = 128
BATCH = 4096
NC = 2
BB = 32
NB = BATCH // BB
NSEM = 8


def _sc_gather_body(table_hbm, idx_hbm, emb_hbm, idx_v, rows_v, gsem):
    wid = lax.axis_index("s") * NC + lax.axis_index("c")

    @pl.when(wid == 0)
    def _():
        pltpu.sync_copy(idx_hbm, idx_v)
        pltpu.make_async_copy(
            table_hbm.at[idx_v.at[pl.ds(0, 128)]], rows_v.at[pl.ds(0, 128)], gsem
        ).start()
        pltpu.make_async_copy(
            table_hbm.at[idx_v.at[pl.ds(128, 72)]], rows_v.at[pl.ds(128, 72)], gsem
        ).start()
        pltpu.make_async_copy(
            table_hbm.at[idx_v.at[pl.ds(0, 128)]], rows_v.at[pl.ds(0, 128)], gsem
        ).wait()
        pltpu.make_async_copy(
            table_hbm.at[idx_v.at[pl.ds(128, 72)]], rows_v.at[pl.ds(128, 72)], gsem
        ).wait()
        pltpu.sync_copy(rows_v, emb_hbm)


def _tc_bcast_body(emb_ref, out_ref, scratch, sems):
    scratch[...] = jnp.broadcast_to(emb_ref[...][None], (BB, SEQ_LEN, D_MODEL))

    def _start(k, c):
        pltpu.make_async_copy(
            scratch, out_ref.at[pl.ds(k * BB, BB)], sems.at[k % NSEM]
        ).start()
        return c

    lax.fori_loop(0, NB, _start, None)

    def _wait(k, c):
        pltpu.make_async_copy(
            scratch, out_ref.at[pl.ds(k * BB, BB)], sems.at[k % NSEM]
        ).wait()
        return c

    lax.fori_loop(0, NB, _wait, None)


@jax.jit
def _run(pos_embed, positions):
    idx = positions.astype(jnp.int32)
    sc_gather = pl.kernel(
        _sc_gather_body,
        out_type=jax.ShapeDtypeStruct((SEQ_LEN, D_MODEL), jnp.float32),
        mesh=plsc.VectorSubcoreMesh(core_axis_name="c", subcore_axis_name="s"),
        scratch_types=[
            pltpu.VMEM((SEQ_LEN,), jnp.int32),
            pltpu.VMEM((SEQ_LEN, D_MODEL), jnp.float32),
            pltpu.SemaphoreType.DMA,
        ],
    )
    emb = sc_gather(pos_embed, idx)
    return pl.pallas_call(
        _tc_bcast_body,
        grid=(1,),
        in_specs=[
            pl.BlockSpec((SEQ_LEN, D_MODEL), lambda i: (0, 0)),
        ],
        out_specs=pl.BlockSpec(memory_space=pl.ANY),
        out_shape=jax.ShapeDtypeStruct((BATCH, SEQ_LEN, D_MODEL), jnp.float32),
        scratch_shapes=[
            pltpu.VMEM((BB, SEQ_LEN, D_MODEL), jnp.float32),
            pltpu.SemaphoreType.DMA((NSEM,)),
        ],
        compiler_params=pltpu.CompilerParams(
            dimension_semantics=("arbitrary",),
        ),
    )(emb)


def kernel(batch_size, pos_embed, positions):
    return _run(pos_embed, positions)


# speedup vs baseline: 1.2409x; 1.1614x over previous
import jax
import jax.numpy as jnp
from jax import lax
from jax.experimental import pallas as pl
from jax.experimental.pallas import tpu as pltpu

SEQ_LEN = 200
D_MODEL = 128
BATCH = 4096
BB = 32
NB = BATCH // BB


def _bcast_kernel(pos_ref, idx_ref, out_ref, scratch, sem):
    pos = idx_ref[...][:, 0]
    onehot = (
        pos[:, None]
        == lax.broadcasted_iota(jnp.int32, (SEQ_LEN, SEQ_LEN), 1)
    ).astype(jnp.float32)
    emb = lax.dot_general(
        onehot,
        pos_ref[...],
        dimension_numbers=(((1,), (0,)), ((), ())),
        preferred_element_type=jnp.float32,
    )
    scratch[...] = jnp.broadcast_to(emb[None], (BB, SEQ_LEN, D_MODEL))

    def _start(k, c):
        pltpu.make_async_copy(
            scratch, out_ref.at[pl.ds(k * BB, BB)], sem
        ).start()
        return c

    lax.fori_loop(0, NB, _start, None)

    pltpu.make_async_copy(out_ref, out_ref, sem).wait()


@jax.jit
def _run(pos_embed, positions):
    idx2d = positions.astype(jnp.int32).reshape(SEQ_LEN, 1)
    return pl.pallas_call(
        _bcast_kernel,
        grid=(1,),
        in_specs=[
            pl.BlockSpec((SEQ_LEN, D_MODEL), lambda i: (0, 0)),
            pl.BlockSpec((SEQ_LEN, 1), lambda i: (0, 0)),
        ],
        out_specs=pl.BlockSpec(memory_space=pl.ANY),
        out_shape=jax.ShapeDtypeStruct((BATCH, SEQ_LEN, D_MODEL), jnp.float32),
        scratch_shapes=[
            pltpu.VMEM((BB, SEQ_LEN, D_MODEL), jnp.float32),
            pltpu.SemaphoreType.DMA,
        ],
        compiler_params=pltpu.CompilerParams(
            dimension_semantics=("arbitrary",),
        ),
    )(pos_embed, idx2d)


def kernel(batch_size, pos_embed, positions):
    return _run(pos_embed, positions)
